# baseline (speedup 1.0000x reference)
import functools

import jax
import jax.numpy as jnp
from jax import lax
from jax.experimental import pallas as pl
from jax.experimental.pallas import tpu as pltpu
from jax.experimental.pallas import tpu_sc as plsc

B = 16
C_IN = 3
IMG = 224
PATCH = 16
GRID = IMG // PATCH
N_PATCH = GRID * GRID
PATCH_DIM = C_IN * PATCH * PATCH
CODE_DIM = 256
NUM_ING = 8192
NUM_CLASSES = 1000

M = B * N_PATCH

BN = 1024
NJ = NUM_ING // BN


def _codes_body(patches_ref, wenc_ref, cb_ref, codes_ref, z_ref, bestv_ref,
                besti_ref):
    j = pl.program_id(0)

    @pl.when(j == 0)
    def _init():
        z_ref[...] = jnp.dot(patches_ref[...], wenc_ref[...],
                             preferred_element_type=jnp.float32)
        bestv_ref[...] = jnp.full_like(bestv_ref[...], -jnp.inf)
        besti_ref[...] = jnp.zeros_like(besti_ref[...])

    cb = cb_ref[...]
    cn = jnp.sum(cb * cb, axis=1)
    s = lax.dot_general(z_ref[...], cb, (((1,), (1,)), ((), ())),
                        preferred_element_type=jnp.float32)
    s = s - 0.5 * cn[None, :]
    lv = jnp.max(s, axis=1)
    idx = lax.broadcasted_iota(jnp.int32, s.shape, 1)
    li = jnp.min(jnp.where(s == lv[:, None], idx, NUM_ING), axis=1)
    li = li + j * BN
    upd = lv > bestv_ref[...]
    bestv_ref[...] = jnp.where(upd, lv, bestv_ref[...])
    besti_ref[...] = jnp.where(upd, li, besti_ref[...])

    @pl.when(j == NJ - 1)
    def _finish():
        codes_ref[...] = besti_ref[...]


def _codes_call(patches, W_enc, codebook):
    return pl.pallas_call(
        _codes_body,
        grid=(NJ,),
        in_specs=[
            pl.BlockSpec((M, PATCH_DIM), lambda j: (0, 0)),
            pl.BlockSpec((PATCH_DIM, CODE_DIM), lambda j: (0, 0)),
            pl.BlockSpec((BN, CODE_DIM), lambda j: (j, 0)),
        ],
        out_specs=pl.BlockSpec((M,), lambda j: (0,)),
        out_shape=jax.ShapeDtypeStruct((M,), jnp.int32),
        scratch_shapes=[
            pltpu.VMEM((M, CODE_DIM), jnp.float32),
            pltpu.VMEM((M,), jnp.float32),
            pltpu.VMEM((M,), jnp.int32),
        ],
    )(patches, W_enc, codebook)


PADN = 224
HALF = PADN // 2
NBINS = NUM_ING + 16
SAMPLES_PER_CORE = B // 2


def _bincount_body(codes_hbm, feat_hbm, idx_v, ones_v, stage_v, bins_sh):
    c = lax.axis_index("c")
    s = lax.axis_index("s")
    b = c * SAMPLES_PER_CORE + s // 2
    b_local = s // 2
    h = s % 2

    pltpu.sync_copy(codes_hbm.at[pl.ds(b * PADN + h * HALF, HALF)], idx_v)
    off = b_local * NBINS
    for i in range(HALF // 16):
        sl = pl.ds(i * 16, 16)
        idx_v[sl] = idx_v[sl] + off
        ones_v[sl] = jnp.full((16,), 1.0, jnp.float32)

    @pl.when(h == 0)
    def _zero():
        def zloop(i, _):
            stage_v[pl.ds(i * 16, 16)] = jnp.zeros((16,), jnp.float32)
            return 0
        lax.fori_loop(0, NBINS // 16, zloop, 0)
        pltpu.sync_copy(stage_v, bins_sh.at[pl.ds(b_local * NBINS, NBINS)])

    plsc.subcore_barrier()

    pltpu.sync_copy(ones_v, bins_sh.at[idx_v], add=True)

    plsc.subcore_barrier()

    @pl.when(h == 0)
    def _writeback():
        pltpu.sync_copy(bins_sh.at[pl.ds(b_local * NBINS, NUM_ING)],
                        stage_v.at[pl.ds(0, NUM_ING)])
        pltpu.sync_copy(stage_v.at[pl.ds(0, NUM_ING)],
                        feat_hbm.at[pl.ds(b * NUM_ING, NUM_ING)])


def _bincount_call(codes_flat):
    mesh = plsc.VectorSubcoreMesh(core_axis_name="c", subcore_axis_name="s")
    fn = functools.partial(
        pl.kernel,
        mesh=mesh,
        out_type=jax.ShapeDtypeStruct((B * NUM_ING,), jnp.float32),
        scratch_types=[
            pltpu.VMEM((HALF,), jnp.int32),
            pltpu.VMEM((HALF,), jnp.float32),
            pltpu.VMEM((NBINS,), jnp.float32),
            pltpu.VMEM_SHARED((SAMPLES_PER_CORE * NBINS,), jnp.float32),
        ],
    )(_bincount_body)
    return fn(codes_flat)


BK = 1024
NK = NUM_ING // BK


def _pred_body(f_ref, w_ref, b_ref, out_ref, acc_ref):
    k = pl.program_id(0)

    @pl.when(k == 0)
    def _init():
        acc_ref[...] = jnp.zeros_like(acc_ref[...])

    acc_ref[...] += lax.dot_general(f_ref[...], w_ref[...],
                                    (((1,), (1,)), ((), ())),
                                    preferred_element_type=jnp.float32)

    @pl.when(k == NK - 1)
    def _finish():
        out_ref[...] = acc_ref[...] + b_ref[...]


def _pred_call(features, fc_W, fc_b):
    return pl.pallas_call(
        _pred_body,
        grid=(NK,),
        in_specs=[
            pl.BlockSpec((B, BK), lambda k: (0, k)),
            pl.BlockSpec((NUM_CLASSES, BK), lambda k: (0, k)),
            pl.BlockSpec((1, NUM_CLASSES), lambda k: (0, 0)),
        ],
        out_specs=pl.BlockSpec((B, NUM_CLASSES), lambda k: (0, 0)),
        out_shape=jax.ShapeDtypeStruct((B, NUM_CLASSES), jnp.float32),
        scratch_shapes=[pltpu.VMEM((B, NUM_CLASSES), jnp.float32)],
    )(features, fc_W, fc_b.reshape(1, NUM_CLASSES))


def kernel(x, W_enc, codebook, fc_W, fc_b):
    patches = x.reshape(B, C_IN, GRID, PATCH, GRID, PATCH)
    patches = patches.transpose(0, 2, 4, 1, 3, 5).reshape(M, PATCH_DIM)
    codes = _codes_call(patches, W_enc, codebook)
    codes2d = jnp.pad(codes.reshape(B, N_PATCH),
                      ((0, 0), (0, PADN - N_PATCH)),
                      constant_values=NUM_ING)
    features = _bincount_call(codes2d.reshape(-1)).reshape(B, NUM_ING)
    pred = _pred_call(features, fc_W, fc_b)
    return (pred, jnp.array(0), jnp.array(0))


# speedup vs baseline: 1.1144x; 1.1144x over previous
import functools

import jax
import jax.numpy as jnp
from jax import lax
from jax.experimental import pallas as pl
from jax.experimental.pallas import tpu as pltpu
from jax.experimental.pallas import tpu_sc as plsc

B = 16
C_IN = 3
IMG = 224
PATCH = 16
GRID = IMG // PATCH
N_PATCH = GRID * GRID
PADN = 208
PATCH_DIM = C_IN * PATCH * PATCH
CODE_DIM = 256
NUM_ING = 8192
NUM_CLASSES = 1000

MP = B * PADN

BN = 1024
NJ = NUM_ING // BN


def _codes_body(patches_ref, wenc_ref, cb_ref, codes_ref, z_ref, bestv_ref,
                besti_ref):
    j = pl.program_id(0)

    @pl.when(j == 0)
    def _init():
        z = jnp.dot(patches_ref[...], wenc_ref[...],
                    preferred_element_type=jnp.float32)
        z_ref[...] = z.astype(jnp.bfloat16)
        bestv_ref[...] = jnp.full_like(bestv_ref[...], -jnp.inf)
        besti_ref[...] = jnp.zeros_like(besti_ref[...])

    cb = cb_ref[...]
    cn = jnp.sum(cb * cb, axis=1)
    s = lax.dot_general(z_ref[...], cb.astype(jnp.bfloat16),
                        (((1,), (1,)), ((), ())),
                        preferred_element_type=jnp.float32)
    s = s - 0.5 * cn[None, :]
    lv = jnp.max(s, axis=1)
    idx = lax.broadcasted_iota(jnp.int32, s.shape, 1)
    li = jnp.min(jnp.where(s == lv[:, None], idx, NUM_ING), axis=1)
    li = li + j * BN
    upd = lv > bestv_ref[...]
    bestv_ref[...] = jnp.where(upd, lv, bestv_ref[...])
    besti_ref[...] = jnp.where(upd, li, besti_ref[...])

    @pl.when(j == NJ - 1)
    def _finish():
        codes_ref[...] = besti_ref[...]


def _codes_call(patches, W_enc, codebook):
    return pl.pallas_call(
        _codes_body,
        grid=(NJ,),
        in_specs=[
            pl.BlockSpec((MP, PATCH_DIM), lambda j: (0, 0)),
            pl.BlockSpec((PATCH_DIM, CODE_DIM), lambda j: (0, 0)),
            pl.BlockSpec((BN, CODE_DIM), lambda j: (j, 0)),
        ],
        out_specs=pl.BlockSpec((MP,), lambda j: (0,)),
        out_shape=jax.ShapeDtypeStruct((MP,), jnp.int32),
        scratch_shapes=[
            pltpu.VMEM((MP, CODE_DIM), jnp.bfloat16),
            pltpu.VMEM((MP,), jnp.float32),
            pltpu.VMEM((MP,), jnp.int32),
        ],
    )(patches, W_enc, codebook)


HALF = PADN // 2
WIN = 112
NBINS = NUM_ING + 16
SAMPLES_PER_CORE = B // 2


def _bincount_body(codes_hbm, feat_hbm, idx_v, ones_v, stage_v, bins_sh):
    c = lax.axis_index("c")
    s = lax.axis_index("s")
    b = c * SAMPLES_PER_CORE + s // 2
    b_local = s // 2
    h = s % 2

    pltpu.sync_copy(codes_hbm.at[pl.ds(b * PADN + h * 96, WIN)], idx_v)
    off = b_local * NBINS
    lo = h * 8
    hi = 104 - 4 * h
    for i in range(WIN // 16):
        sl = pl.ds(i * 16, 16)
        rel = lax.iota(jnp.int32, 16) + (i * 16)
        ok = (rel >= lo) & (rel < hi)
        idx_v[sl] = jnp.where(ok, idx_v[sl], NUM_ING) + off
        ones_v[sl] = jnp.full((16,), 1.0, jnp.float32)

    @pl.when(h == 0)
    def _zero():
        def zloop(i, _):
            stage_v[pl.ds(i * 16, 16)] = jnp.zeros((16,), jnp.float32)
            return 0
        lax.fori_loop(0, NBINS // 16, zloop, 0)
        pltpu.sync_copy(stage_v, bins_sh.at[pl.ds(b_local * NBINS, NBINS)])

    plsc.subcore_barrier()

    pltpu.sync_copy(ones_v, bins_sh.at[idx_v], add=True)

    plsc.subcore_barrier()

    @pl.when(h == 0)
    def _writeback():
        pltpu.sync_copy(bins_sh.at[pl.ds(b_local * NBINS, NUM_ING)],
                        stage_v.at[pl.ds(0, NUM_ING)])
        pltpu.sync_copy(stage_v.at[pl.ds(0, NUM_ING)],
                        feat_hbm.at[pl.ds(b * NUM_ING, NUM_ING)])


def _bincount_call(codes_flat):
    mesh = plsc.VectorSubcoreMesh(core_axis_name="c", subcore_axis_name="s")
    fn = functools.partial(
        pl.kernel,
        mesh=mesh,
        out_type=jax.ShapeDtypeStruct((B * NUM_ING,), jnp.float32),
        scratch_types=[
            pltpu.VMEM((WIN,), jnp.int32),
            pltpu.VMEM((WIN,), jnp.float32),
            pltpu.VMEM((NBINS,), jnp.float32),
            pltpu.VMEM_SHARED((SAMPLES_PER_CORE * NBINS,), jnp.float32),
        ],
    )(_bincount_body)
    return fn(codes_flat)


BK = 1024
NK = NUM_ING // BK


def _pred_body(f_ref, w_ref, b_ref, out_ref, acc_ref):
    k = pl.program_id(0)

    @pl.when(k == 0)
    def _init():
        acc_ref[...] = jnp.zeros_like(acc_ref[...])

    acc_ref[...] += lax.dot_general(f_ref[...], w_ref[...],
                                    (((1,), (1,)), ((), ())),
                                    preferred_element_type=jnp.float32)

    @pl.when(k == NK - 1)
    def _finish():
        out_ref[...] = acc_ref[...] + b_ref[...]


def _pred_call(features, fc_W, fc_b):
    return pl.pallas_call(
        _pred_body,
        grid=(NK,),
        in_specs=[
            pl.BlockSpec((B, BK), lambda k: (0, k)),
            pl.BlockSpec((NUM_CLASSES, BK), lambda k: (0, k)),
            pl.BlockSpec((1, NUM_CLASSES), lambda k: (0, 0)),
        ],
        out_specs=pl.BlockSpec((B, NUM_CLASSES), lambda k: (0, 0)),
        out_shape=jax.ShapeDtypeStruct((B, NUM_CLASSES), jnp.float32),
        scratch_shapes=[pltpu.VMEM((B, NUM_CLASSES), jnp.float32)],
    )(features, fc_W, fc_b.reshape(1, NUM_CLASSES))


def kernel(x, W_enc, codebook, fc_W, fc_b):
    xb = x.astype(jnp.bfloat16)
    x6 = xb.reshape(B, C_IN, GRID, PATCH, GRID, PATCH)
    p3 = x6.transpose(0, 2, 4, 1, 3, 5).reshape(B, N_PATCH, PATCH_DIM)
    p3 = jnp.pad(p3, ((0, 0), (0, PADN - N_PATCH), (0, 0)))
    patches = p3.reshape(MP, PATCH_DIM)
    codes = _codes_call(patches, W_enc.astype(jnp.bfloat16), codebook)
    features = _bincount_call(codes).reshape(B, NUM_ING)
    pred = _pred_call(features, fc_W, fc_b)
    return (pred, jnp.array(0), jnp.array(0))


# speedup vs baseline: 1.2414x; 1.1139x over previous
import functools

import jax
import jax.numpy as jnp
from jax import lax
from jax.experimental import pallas as pl
from jax.experimental.pallas import tpu as pltpu
from jax.experimental.pallas import tpu_sc as plsc

B = 16
C_IN = 3
IMG = 224
PATCH = 16
GRID = IMG // PATCH
N_PATCH = GRID * GRID
PATCH_DIM = C_IN * PATCH * PATCH
CODE_DIM = 256
NUM_ING = 8192
NUM_CLASSES = 1000

M = N_PATCH * B

BN = 1024
NJ = NUM_ING // BN


def _codes_body(patches_ref, wenc_ref, cb_ref, codes_ref, z_ref, bestv_ref,
                besti_ref):
    j = pl.program_id(0)

    @pl.when(j == 0)
    def _init():
        p = patches_ref[...].reshape(M, PATCH_DIM)
        z = jnp.dot(p, wenc_ref[...], preferred_element_type=jnp.float32)
        z_ref[...] = z.astype(jnp.bfloat16)
        bestv_ref[...] = jnp.full_like(bestv_ref[...], -jnp.inf)
        besti_ref[...] = jnp.zeros_like(besti_ref[...])

    cb = cb_ref[...]
    cn = jnp.sum(cb * cb, axis=1)
    s = lax.dot_general(z_ref[...], cb.astype(jnp.bfloat16),
                        (((1,), (1,)), ((), ())),
                        preferred_element_type=jnp.float32)
    s = s - 0.5 * cn[None, :]
    lv = jnp.max(s, axis=1)
    idx = lax.broadcasted_iota(jnp.int32, s.shape, 1)
    li = jnp.min(jnp.where(s == lv[:, None], idx, NUM_ING), axis=1)
    li = li + j * BN
    upd = lv > bestv_ref[...]
    bestv_ref[...] = jnp.where(upd, lv, bestv_ref[...])
    besti_ref[...] = jnp.where(upd, li, besti_ref[...])

    @pl.when(j == NJ - 1)
    def _finish():
        codes_ref[...] = besti_ref[...]


def _codes_call(patches3, W_enc, codebook):
    return pl.pallas_call(
        _codes_body,
        grid=(NJ,),
        in_specs=[
            pl.BlockSpec((N_PATCH, B, PATCH_DIM), lambda j: (0, 0, 0)),
            pl.BlockSpec((PATCH_DIM, CODE_DIM), lambda j: (0, 0)),
            pl.BlockSpec((BN, CODE_DIM), lambda j: (j, 0)),
        ],
        out_specs=pl.BlockSpec((M,), lambda j: (0,)),
        out_shape=jax.ShapeDtypeStruct((M,), jnp.int32),
        scratch_shapes=[
            pltpu.VMEM((M, CODE_DIM), jnp.bfloat16),
            pltpu.VMEM((M,), jnp.float32),
            pltpu.VMEM((M,), jnp.int32),
        ],
    )(patches3, W_enc, codebook)


NCHUNK = 7
NBINS = NUM_ING + 16
NTILE = 32


def _bincount_body(codes_hbm, feat_hbm, win_v, idx_v, ones_v, stage_v,
                   bins_sh):
    c = lax.axis_index("c")
    s = lax.axis_index("s")
    t = c * 16 + s
    has7 = t >= 28
    start = jnp.where(has7, 168 + 7 * (t - 28), 6 * t)

    pltpu.sync_copy(codes_hbm.at[pl.ds(start * 16, NCHUNK * 16)], win_v)
    lane = lax.iota(jnp.int32, 16)
    for k in range(NCHUNK):
        sl = pl.ds(k * 16, 16)
        idx_v[sl] = lane * NBINS + NUM_ING
        ones_v[sl] = jnp.full((16,), 1.0, jnp.float32)
        if k < 6:
            idx_v[sl] = lane * NBINS + win_v[sl]
        else:
            @pl.when(has7)
            def _last():
                idx_v[sl] = lane * NBINS + win_v[sl]

    def zloop(i, _):
        stage_v[pl.ds(i * 16, 16)] = jnp.zeros((16,), jnp.float32)
        return 0
    lax.fori_loop(0, NBINS // 16, zloop, 0)
    pltpu.sync_copy(stage_v, bins_sh.at[pl.ds(s * NBINS, NBINS)])

    plsc.subcore_barrier()

    pltpu.sync_copy(ones_v, bins_sh.at[idx_v], add=True)

    plsc.subcore_barrier()

    pltpu.sync_copy(bins_sh.at[pl.ds(s * NBINS, NUM_ING)],
                    stage_v.at[pl.ds(0, NUM_ING)])
    pltpu.sync_copy(stage_v.at[pl.ds(0, NUM_ING)],
                    feat_hbm.at[pl.ds((c * B + s) * NUM_ING, NUM_ING)])


def _bincount_call(codes_flat):
    mesh = plsc.VectorSubcoreMesh(core_axis_name="c", subcore_axis_name="s")
    fn = functools.partial(
        pl.kernel,
        mesh=mesh,
        out_type=jax.ShapeDtypeStruct((2 * B * NUM_ING,), jnp.float32),
        scratch_types=[
            pltpu.VMEM((NCHUNK * 16,), jnp.int32),
            pltpu.VMEM((NCHUNK * 16,), jnp.int32),
            pltpu.VMEM((NCHUNK * 16,), jnp.float32),
            pltpu.VMEM((NBINS,), jnp.float32),
            pltpu.VMEM_SHARED((B * NBINS,), jnp.float32),
        ],
    )(_bincount_body)
    return fn(codes_flat)


BK = 1024
NK = NUM_ING // BK


def _pred_body(f_ref, w_ref, b_ref, out_ref, acc_ref):
    k = pl.program_id(0)

    @pl.when(k == 0)
    def _init():
        acc_ref[...] = jnp.zeros_like(acc_ref[...])

    f = f_ref[0] + f_ref[1]
    acc_ref[...] += lax.dot_general(f, w_ref[...],
                                    (((1,), (1,)), ((), ())),
                                    preferred_element_type=jnp.float32)

    @pl.when(k == NK - 1)
    def _finish():
        out_ref[...] = acc_ref[...] + b_ref[...]


def _pred_call(features, fc_W, fc_b):
    return pl.pallas_call(
        _pred_body,
        grid=(NK,),
        in_specs=[
            pl.BlockSpec((2, B, BK), lambda k: (0, 0, k)),
            pl.BlockSpec((NUM_CLASSES, BK), lambda k: (0, k)),
            pl.BlockSpec((1, NUM_CLASSES), lambda k: (0, 0)),
        ],
        out_specs=pl.BlockSpec((B, NUM_CLASSES), lambda k: (0, 0)),
        out_shape=jax.ShapeDtypeStruct((B, NUM_CLASSES), jnp.float32),
        scratch_shapes=[pltpu.VMEM((B, NUM_CLASSES), jnp.float32)],
    )(features, fc_W, fc_b.reshape(1, NUM_CLASSES))


def kernel(x, W_enc, codebook, fc_W, fc_b):
    xb = x.astype(jnp.bfloat16)
    x6 = xb.reshape(B, C_IN, GRID, PATCH, GRID, PATCH)
    patches3 = x6.transpose(2, 4, 0, 1, 3, 5).reshape(N_PATCH, B, PATCH_DIM)
    codes = _codes_call(patches3, W_enc.astype(jnp.bfloat16), codebook)
    features = _bincount_call(codes).reshape(2, B, NUM_ING)
    pred = _pred_call(features, fc_W, fc_b)
    return (pred, jnp.array(0), jnp.array(0))


# speedup vs baseline: 1.2551x; 1.0110x over previous
import functools

import jax
import jax.numpy as jnp
from jax import lax
from jax.experimental import pallas as pl
from jax.experimental.pallas import tpu as pltpu
from jax.experimental.pallas import tpu_sc as plsc

B = 16
C_IN = 3
IMG = 224
PATCH = 16
GRID = IMG // PATCH
N_PATCH = GRID * GRID
PATCH_DIM = C_IN * PATCH * PATCH
CODE_DIM = 256
NUM_ING = 8192
NUM_CLASSES = 1000

M = N_PATCH * B

BN = 2048
NJ = NUM_ING // BN


def _codes_body(patches_ref, wenc_ref, cb_ref, codes_ref, z_ref, bestv_ref,
                besti_ref):
    j = pl.program_id(0)

    @pl.when(j == 0)
    def _init():
        p = patches_ref[...].reshape(M, PATCH_DIM)
        z = jnp.dot(p, wenc_ref[...], preferred_element_type=jnp.float32)
        z_ref[...] = z.astype(jnp.bfloat16)
        bestv_ref[...] = jnp.full_like(bestv_ref[...], -jnp.inf)
        besti_ref[...] = jnp.zeros_like(besti_ref[...])

    cb = cb_ref[...]
    cn = jnp.sum(cb * cb, axis=1)
    s = lax.dot_general(z_ref[...], cb.astype(jnp.bfloat16),
                        (((1,), (1,)), ((), ())),
                        preferred_element_type=jnp.float32)
    s = s - 0.5 * cn[None, :]
    lv = jnp.max(s, axis=1)
    idx = lax.broadcasted_iota(jnp.int32, s.shape, 1)
    li = jnp.min(jnp.where(s == lv[:, None], idx, NUM_ING), axis=1)
    li = li + j * BN
    upd = lv > bestv_ref[...]
    bestv_ref[...] = jnp.where(upd, lv, bestv_ref[...])
    besti_ref[...] = jnp.where(upd, li, besti_ref[...])

    @pl.when(j == NJ - 1)
    def _finish():
        codes_ref[...] = besti_ref[...]


def _codes_call(patches3, W_enc, codebook):
    return pl.pallas_call(
        _codes_body,
        grid=(NJ,),
        in_specs=[
            pl.BlockSpec((N_PATCH, B, PATCH_DIM), lambda j: (0, 0, 0)),
            pl.BlockSpec((PATCH_DIM, CODE_DIM), lambda j: (0, 0)),
            pl.BlockSpec((BN, CODE_DIM), lambda j: (j, 0)),
        ],
        out_specs=pl.BlockSpec((M,), lambda j: (0,)),
        out_shape=jax.ShapeDtypeStruct((M,), jnp.int32),
        scratch_shapes=[
            pltpu.VMEM((M, CODE_DIM), jnp.bfloat16),
            pltpu.VMEM((M,), jnp.float32),
            pltpu.VMEM((M,), jnp.int32),
        ],
    )(patches3, W_enc, codebook)


NCHUNK = 7
NBINS = NUM_ING + 16
NTILE = 32


def _bincount_body(codes_hbm, feat_hbm, win_v, idx_v, ones_v, stage_v,
                   bins_sh):
    c = lax.axis_index("c")
    s = lax.axis_index("s")
    t = c * 16 + s
    has7 = t >= 28
    start = jnp.where(has7, 168 + 7 * (t - 28), 6 * t)

    pltpu.sync_copy(codes_hbm.at[pl.ds(start * 16, NCHUNK * 16)], win_v)
    lane = lax.iota(jnp.int32, 16)
    for k in range(NCHUNK):
        sl = pl.ds(k * 16, 16)
        idx_v[sl] = lane * NBINS + NUM_ING
        ones_v[sl] = jnp.full((16,), 1.0, jnp.float32)
        if k < 6:
            idx_v[sl] = lane * NBINS + win_v[sl]
        else:
            @pl.when(has7)
            def _last():
                idx_v[sl] = lane * NBINS + win_v[sl]

    def zloop(i, _):
        stage_v[pl.ds(i * 16, 16)] = jnp.zeros((16,), jnp.float32)
        return 0
    lax.fori_loop(0, NBINS // 16, zloop, 0)
    pltpu.sync_copy(stage_v, bins_sh.at[pl.ds(s * NBINS, NBINS)])

    plsc.subcore_barrier()

    pltpu.sync_copy(ones_v, bins_sh.at[idx_v], add=True)

    plsc.subcore_barrier()

    pltpu.sync_copy(bins_sh.at[pl.ds(s * NBINS, NUM_ING)],
                    stage_v.at[pl.ds(0, NUM_ING)])
    pltpu.sync_copy(stage_v.at[pl.ds(0, NUM_ING)],
                    feat_hbm.at[pl.ds((c * B + s) * NUM_ING, NUM_ING)])


def _bincount_call(codes_flat):
    mesh = plsc.VectorSubcoreMesh(core_axis_name="c", subcore_axis_name="s")
    fn = functools.partial(
        pl.kernel,
        mesh=mesh,
        out_type=jax.ShapeDtypeStruct((2 * B * NUM_ING,), jnp.float32),
        scratch_types=[
            pltpu.VMEM((NCHUNK * 16,), jnp.int32),
            pltpu.VMEM((NCHUNK * 16,), jnp.int32),
            pltpu.VMEM((NCHUNK * 16,), jnp.float32),
            pltpu.VMEM((NBINS,), jnp.float32),
            pltpu.VMEM_SHARED((B * NBINS,), jnp.float32),
        ],
    )(_bincount_body)
    return fn(codes_flat)


BK = 1024
NK = NUM_ING // BK


def _pred_body(f_ref, w_ref, b_ref, out_ref, acc_ref):
    k = pl.program_id(0)

    @pl.when(k == 0)
    def _init():
        acc_ref[...] = jnp.zeros_like(acc_ref[...])

    f = f_ref[0] + f_ref[1]
    acc_ref[...] += lax.dot_general(f, w_ref[...],
                                    (((1,), (1,)), ((), ())),
                                    preferred_element_type=jnp.float32)

    @pl.when(k == NK - 1)
    def _finish():
        out_ref[...] = acc_ref[...] + b_ref[...]


def _pred_call(features, fc_W, fc_b):
    return pl.pallas_call(
        _pred_body,
        grid=(NK,),
        in_specs=[
            pl.BlockSpec((2, B, BK), lambda k: (0, 0, k)),
            pl.BlockSpec((NUM_CLASSES, BK), lambda k: (0, k)),
            pl.BlockSpec((1, NUM_CLASSES), lambda k: (0, 0)),
        ],
        out_specs=pl.BlockSpec((B, NUM_CLASSES), lambda k: (0, 0)),
        out_shape=jax.ShapeDtypeStruct((B, NUM_CLASSES), jnp.float32),
        scratch_shapes=[pltpu.VMEM((B, NUM_CLASSES), jnp.float32)],
    )(features, fc_W, fc_b.reshape(1, NUM_CLASSES))


def kernel(x, W_enc, codebook, fc_W, fc_b):
    xb = x.astype(jnp.bfloat16)
    x6 = xb.reshape(B, C_IN, GRID, PATCH, GRID, PATCH)
    patches3 = x6.transpose(2, 4, 0, 1, 3, 5).reshape(N_PATCH, B, PATCH_DIM)
    codes = _codes_call(patches3, W_enc.astype(jnp.bfloat16), codebook)
    features = _bincount_call(codes).reshape(2, B, NUM_ING)
    pred = _pred_call(features, fc_W, fc_b)
    return (pred, jnp.array(0), jnp.array(0))
